# baseline (speedup 1.0000x reference)
import functools

import jax
import jax.numpy as jnp
from jax import lax
from jax.experimental import pallas as pl
from jax.experimental.pallas import tpu as pltpu
from jax.experimental.pallas import tpu_sc as plsc

N_NODES = 10000
N_EDGES = 320000
DIM = 128

_NC = 2
_NS = 16
_NW = _NC * _NS
_CHUNK = 200
_NBUF = 4
_BLK = 16000

_SEG_SIZES = (32000, 32000, 64000, 64000, 64000, 32000, 32000)
_SEG_OFFS = (0, 32000, 64000, 128000, 192000, 256000, 288000)


def _proj_body(n_ref, ws_ref, wd_ref, a_ref, b_ref):
    n = n_ref[...]
    dn = (((1,), (1,)), ((), ()))
    a_ref[...] = lax.dot_general(n, ws_ref[...], dn,
                                 preferred_element_type=jnp.float32)
    b_ref[...] = lax.dot_general(n, wd_ref[...], dn,
                                 preferred_element_type=jnp.float32)


def _node_proj(nfeat, W_s, W_d):
    out_sd = jax.ShapeDtypeStruct((N_NODES, DIM), jnp.float32)
    row = pl.BlockSpec((2000, DIM), lambda i: (i, 0))
    w = pl.BlockSpec((DIM, DIM), lambda i: (0, 0))
    return pl.pallas_call(
        _proj_body,
        grid=(N_NODES // 2000,),
        in_specs=[row, w, w],
        out_specs=(row, row),
        out_shape=(out_sd, out_sd),
    )(nfeat, W_s, W_d)


def _gather_body(seg_base, epw, a_hbm, b_hbm, src_hbm, dst_hbm, gsum_hbm,
                 idx_s, idx_d, r0, r1, r2, r3, s0, s1, s2, s3,
                 w0, w1, w2, w3):
    nchunk = epw // _CHUNK
    wid = lax.axis_index("s") * _NC + lax.axis_index("c")
    wbase = seg_base + wid * epw
    lbase = wid * epw
    bufs, sems = (r0, r1, r2, r3), (s0, s1, s2, s3)
    wsems = (w0, w1, w2, w3)

    pltpu.sync_copy(src_hbm.at[pl.ds(wbase, epw)], idx_s)
    pltpu.sync_copy(dst_hbm.at[pl.ds(wbase, epw)], idx_d)

    def start_a(c, buf):
        pltpu.async_copy(
            a_hbm.at[idx_s.at[pl.ds(c * _CHUNK, _CHUNK)]], bufs[buf],
            sems[buf])

    def start_b_add(c, buf):
        pltpu.async_copy(
            b_hbm.at[idx_d.at[pl.ds(c * _CHUNK, _CHUNK)]], bufs[buf],
            sems[buf], add=True)

    def wait(buf):
        pltpu.make_async_copy(
            a_hbm.at[idx_s.at[pl.ds(0, _CHUNK)]], bufs[buf], sems[buf]).wait()

    def start_wb(c, buf):
        pltpu.async_copy(bufs[buf],
                         gsum_hbm.at[pl.ds(lbase + c * _CHUNK, _CHUNK)],
                         wsems[buf])

    def wait_wb(buf):
        pltpu.make_async_copy(
            bufs[buf], gsum_hbm.at[pl.ds(lbase, _CHUNK)], wsems[buf]).wait()

    start_a(0, 0)
    wait(0)
    start_b_add(0, 0)
    if nchunk > 1:
        start_a(1, 1)
    if nchunk > 2:
        start_a(2, 2)

    @pl.loop(0, nchunk, step=_NBUF)
    def _outer(j):
        for b in range(_NBUF):
            c = j + b
            nb = (b + 1) % _NBUF
            nb3 = (b + 3) % _NBUF

            @pl.when(c < nchunk)
            def _chunk_c():
                wait(b)
                start_wb(c, b)

                @pl.when(c + 1 < nchunk)
                def _():
                    wait(nb)
                    start_b_add(c + 1, nb)

                if b == 0:
                    @pl.when((c + 3 < nchunk) & (c >= 1))
                    def _():
                        wait_wb(nb3)
                        start_a(c + 3, nb3)

                    @pl.when((c + 3 < nchunk) & (c < 1))
                    def _():
                        start_a(c + 3, nb3)
                else:
                    @pl.when(c + 3 < nchunk)
                    def _():
                        wait_wb(nb3)
                        start_a(c + 3, nb3)

    for b in range(_NBUF):
        wait_wb(b)


def _edge_gather(a, b, src, dst, seg):
    size = _SEG_SIZES[seg]
    epw = size // _NW
    out_sd = jax.ShapeDtypeStruct((size, DIM), jnp.float32)
    mesh = plsc.VectorSubcoreMesh(core_axis_name="c", subcore_axis_name="s")
    f = functools.partial(
        pl.kernel,
        out_type=out_sd,
        mesh=mesh,
        scratch_types=[
            pltpu.VMEM((epw,), jnp.int32),
            pltpu.VMEM((epw,), jnp.int32),
            pltpu.VMEM((_CHUNK, DIM), jnp.float32),
            pltpu.VMEM((_CHUNK, DIM), jnp.float32),
            pltpu.VMEM((_CHUNK, DIM), jnp.float32),
            pltpu.VMEM((_CHUNK, DIM), jnp.float32),
            pltpu.SemaphoreType.DMA,
            pltpu.SemaphoreType.DMA,
            pltpu.SemaphoreType.DMA,
            pltpu.SemaphoreType.DMA,
            pltpu.SemaphoreType.DMA,
            pltpu.SemaphoreType.DMA,
            pltpu.SemaphoreType.DMA,
            pltpu.SemaphoreType.DMA,
        ],
    )(functools.partial(_gather_body, _SEG_OFFS[seg], epw))
    return f(a, b, src, dst)


def _edge_mlp_body(e_ref, g_ref, we_ref, b1_ref, wo_ref, bo_ref,
                   gm_ref, bt_ref, out_ref):
    e = e_ref[...]
    dn = (((1,), (1,)), ((), ()))
    h = lax.dot_general(e, we_ref[...], dn, preferred_element_type=jnp.float32)
    h = h + g_ref[...] + b1_ref[...]
    h = h * (1.0 / (1.0 + jnp.exp(-h)))
    o = lax.dot_general(h, wo_ref[...], dn, preferred_element_type=jnp.float32)
    o = o + bo_ref[...]
    mean = jnp.mean(o, axis=-1, keepdims=True)
    cen = o - mean
    var = jnp.mean(cen * cen, axis=-1, keepdims=True)
    o = cen * lax.rsqrt(var + 1e-5) * gm_ref[...] + bt_ref[...]
    out_ref[...] = o + e


def _edge_mlp_seg_first(e_ref, g_ref, *rest):
    _edge_mlp_body(e_ref, g_ref, *rest)


def _edge_mlp_seg_chain(_buf_ref, e_ref, g_ref, *rest):
    _edge_mlp_body(e_ref, g_ref, *rest)


def _edge_mlp(prev, efeat, gsum_seg, seg, W_e, b1, W_out, b_out, gamma, beta):
    blk_off = _SEG_OFFS[seg] // _BLK
    steps = _SEG_SIZES[seg] // _BLK
    seg_row = pl.BlockSpec((_BLK, DIM), lambda i: (blk_off + i, 0))
    loc_row = pl.BlockSpec((_BLK, DIM), lambda i: (i, 0))
    w_spec = pl.BlockSpec((DIM, DIM), lambda i: (0, 0))
    v_spec = pl.BlockSpec((1, DIM), lambda i: (0, 0))
    any_spec = pl.BlockSpec(memory_space=pl.ANY)
    common = [loc_row, w_spec, v_spec, w_spec, v_spec, v_spec, v_spec]
    args = (efeat, gsum_seg, W_e, b1.reshape(1, DIM), W_out,
            b_out.reshape(1, DIM), gamma.reshape(1, DIM), beta.reshape(1, DIM))
    if prev is None:
        return pl.pallas_call(
            _edge_mlp_seg_first,
            grid=(steps,),
            in_specs=[seg_row] + common,
            out_specs=seg_row,
            out_shape=jax.ShapeDtypeStruct((N_EDGES, DIM), jnp.float32),
        )(*args)
    return pl.pallas_call(
        _edge_mlp_seg_chain,
        grid=(steps,),
        in_specs=[any_spec, seg_row] + common,
        out_specs=seg_row,
        out_shape=jax.ShapeDtypeStruct((N_EDGES, DIM), jnp.float32),
        input_output_aliases={0: 0},
    )(prev, *args)


def kernel(efeat, nfeat, src, dst, W_e, W_s, W_d, b1, W_out, b_out, gamma, beta):
    a, b = _node_proj(nfeat, W_s, W_d)
    nseg = len(_SEG_SIZES)
    gsums = [_edge_gather(a, b, src, dst, seg) for seg in range(nseg)]
    out = None
    for seg in range(nseg):
        out = _edge_mlp(out, efeat, gsums[seg], seg,
                        W_e, b1, W_out, b_out, gamma, beta)
    return (out, nfeat)


# speedup vs baseline: 1.0196x; 1.0196x over previous
import functools

import jax
import jax.numpy as jnp
from jax import lax
from jax.experimental import pallas as pl
from jax.experimental.pallas import tpu as pltpu
from jax.experimental.pallas import tpu_sc as plsc

N_NODES = 10000
N_EDGES = 320000
DIM = 128

_NC = 2
_NS = 16
_NW = _NC * _NS
_CHUNK = 200
_NBUF = 4
_BLK = 16000

_SEG_SIZES = (32000, 32000, 64000, 64000, 64000, 64000)
_SEG_OFFS = (0, 32000, 64000, 128000, 192000, 256000)


def _proj_body(n_ref, ws_ref, wd_ref, a_ref, b_ref):
    n = n_ref[...]
    dn = (((1,), (1,)), ((), ()))
    a_ref[...] = lax.dot_general(n, ws_ref[...], dn,
                                 preferred_element_type=jnp.float32)
    b_ref[...] = lax.dot_general(n, wd_ref[...], dn,
                                 preferred_element_type=jnp.float32)


def _node_proj(nfeat, W_s, W_d):
    out_sd = jax.ShapeDtypeStruct((N_NODES, DIM), jnp.float32)
    row = pl.BlockSpec((2000, DIM), lambda i: (i, 0))
    w = pl.BlockSpec((DIM, DIM), lambda i: (0, 0))
    return pl.pallas_call(
        _proj_body,
        grid=(N_NODES // 2000,),
        in_specs=[row, w, w],
        out_specs=(row, row),
        out_shape=(out_sd, out_sd),
    )(nfeat, W_s, W_d)


def _gather_body(seg_base, epw, a_hbm, b_hbm, src_hbm, dst_hbm, gsum_hbm,
                 idx_s, idx_d, r0, r1, r2, r3, s0, s1, s2, s3,
                 w0, w1, w2, w3):
    nchunk = epw // _CHUNK
    wid = lax.axis_index("s") * _NC + lax.axis_index("c")
    wbase = seg_base + wid * epw
    lbase = wid * epw
    bufs, sems = (r0, r1, r2, r3), (s0, s1, s2, s3)
    wsems = (w0, w1, w2, w3)

    pltpu.sync_copy(src_hbm.at[pl.ds(wbase, epw)], idx_s)
    pltpu.sync_copy(dst_hbm.at[pl.ds(wbase, epw)], idx_d)

    def start_a(c, buf):
        pltpu.async_copy(
            a_hbm.at[idx_s.at[pl.ds(c * _CHUNK, _CHUNK)]], bufs[buf],
            sems[buf])

    def start_b_add(c, buf):
        pltpu.async_copy(
            b_hbm.at[idx_d.at[pl.ds(c * _CHUNK, _CHUNK)]], bufs[buf],
            sems[buf], add=True)

    def wait(buf):
        pltpu.make_async_copy(
            a_hbm.at[idx_s.at[pl.ds(0, _CHUNK)]], bufs[buf], sems[buf]).wait()

    def start_wb(c, buf):
        pltpu.async_copy(bufs[buf],
                         gsum_hbm.at[pl.ds(lbase + c * _CHUNK, _CHUNK)],
                         wsems[buf])

    def wait_wb(buf):
        pltpu.make_async_copy(
            bufs[buf], gsum_hbm.at[pl.ds(lbase, _CHUNK)], wsems[buf]).wait()

    start_a(0, 0)
    wait(0)
    start_b_add(0, 0)
    if nchunk > 1:
        start_a(1, 1)
    if nchunk > 2:
        start_a(2, 2)

    @pl.loop(0, nchunk, step=_NBUF)
    def _outer(j):
        for b in range(_NBUF):
            c = j + b
            nb = (b + 1) % _NBUF
            nb3 = (b + 3) % _NBUF

            @pl.when(c < nchunk)
            def _chunk_c():
                wait(b)
                start_wb(c, b)

                @pl.when(c + 1 < nchunk)
                def _():
                    wait(nb)
                    start_b_add(c + 1, nb)

                if b == 0:
                    @pl.when((c + 3 < nchunk) & (c >= 1))
                    def _():
                        wait_wb(nb3)
                        start_a(c + 3, nb3)

                    @pl.when((c + 3 < nchunk) & (c < 1))
                    def _():
                        start_a(c + 3, nb3)
                else:
                    @pl.when(c + 3 < nchunk)
                    def _():
                        wait_wb(nb3)
                        start_a(c + 3, nb3)

    for b in range(_NBUF):
        wait_wb(b)


def _edge_gather(a, b, src, dst, seg):
    size = _SEG_SIZES[seg]
    epw = size // _NW
    out_sd = jax.ShapeDtypeStruct((size, DIM), jnp.float32)
    mesh = plsc.VectorSubcoreMesh(core_axis_name="c", subcore_axis_name="s")
    f = functools.partial(
        pl.kernel,
        out_type=out_sd,
        mesh=mesh,
        scratch_types=[
            pltpu.VMEM((epw,), jnp.int32),
            pltpu.VMEM((epw,), jnp.int32),
            pltpu.VMEM((_CHUNK, DIM), jnp.float32),
            pltpu.VMEM((_CHUNK, DIM), jnp.float32),
            pltpu.VMEM((_CHUNK, DIM), jnp.float32),
            pltpu.VMEM((_CHUNK, DIM), jnp.float32),
            pltpu.SemaphoreType.DMA,
            pltpu.SemaphoreType.DMA,
            pltpu.SemaphoreType.DMA,
            pltpu.SemaphoreType.DMA,
            pltpu.SemaphoreType.DMA,
            pltpu.SemaphoreType.DMA,
            pltpu.SemaphoreType.DMA,
            pltpu.SemaphoreType.DMA,
        ],
    )(functools.partial(_gather_body, _SEG_OFFS[seg], epw))
    return f(a, b, src, dst)


def _edge_mlp_body(e_ref, g_ref, we_ref, b1_ref, wo_ref, bo_ref,
                   gm_ref, bt_ref, out_ref):
    e = e_ref[...]
    dn = (((1,), (1,)), ((), ()))
    h = lax.dot_general(e, we_ref[...], dn, preferred_element_type=jnp.float32)
    h = h + g_ref[...] + b1_ref[...]
    h = h * (1.0 / (1.0 + jnp.exp(-h)))
    o = lax.dot_general(h, wo_ref[...], dn, preferred_element_type=jnp.float32)
    o = o + bo_ref[...]
    mean = jnp.mean(o, axis=-1, keepdims=True)
    cen = o - mean
    var = jnp.mean(cen * cen, axis=-1, keepdims=True)
    o = cen * lax.rsqrt(var + 1e-5) * gm_ref[...] + bt_ref[...]
    out_ref[...] = o + e


def _edge_mlp_seg_first(e_ref, g_ref, *rest):
    _edge_mlp_body(e_ref, g_ref, *rest)


def _edge_mlp_seg_chain(_buf_ref, e_ref, g_ref, *rest):
    _edge_mlp_body(e_ref, g_ref, *rest)


def _edge_mlp(prev, efeat, gsum_seg, seg, W_e, b1, W_out, b_out, gamma, beta):
    blk_off = _SEG_OFFS[seg] // _BLK
    steps = _SEG_SIZES[seg] // _BLK
    seg_row = pl.BlockSpec((_BLK, DIM), lambda i: (blk_off + i, 0))
    loc_row = pl.BlockSpec((_BLK, DIM), lambda i: (i, 0))
    w_spec = pl.BlockSpec((DIM, DIM), lambda i: (0, 0))
    v_spec = pl.BlockSpec((1, DIM), lambda i: (0, 0))
    any_spec = pl.BlockSpec(memory_space=pl.ANY)
    common = [loc_row, w_spec, v_spec, w_spec, v_spec, v_spec, v_spec]
    args = (efeat, gsum_seg, W_e, b1.reshape(1, DIM), W_out,
            b_out.reshape(1, DIM), gamma.reshape(1, DIM), beta.reshape(1, DIM))
    if prev is None:
        return pl.pallas_call(
            _edge_mlp_seg_first,
            grid=(steps,),
            in_specs=[seg_row] + common,
            out_specs=seg_row,
            out_shape=jax.ShapeDtypeStruct((N_EDGES, DIM), jnp.float32),
        )(*args)
    return pl.pallas_call(
        _edge_mlp_seg_chain,
        grid=(steps,),
        in_specs=[any_spec, seg_row] + common,
        out_specs=seg_row,
        out_shape=jax.ShapeDtypeStruct((N_EDGES, DIM), jnp.float32),
        input_output_aliases={0: 0},
    )(prev, *args)


def kernel(efeat, nfeat, src, dst, W_e, W_s, W_d, b1, W_out, b_out, gamma, beta):
    a, b = _node_proj(nfeat, W_s, W_d)
    nseg = len(_SEG_SIZES)
    gsums = [_edge_gather(a, b, src, dst, seg) for seg in range(nseg)]
    out = None
    for seg in range(nseg):
        out = _edge_mlp(out, efeat, gsums[seg], seg,
                        W_e, b1, W_out, b_out, gamma, beta)
    return (out, nfeat)
